# Initial kernel scaffold; baseline (speedup 1.0000x reference)
import jax
import jax.numpy as jnp
from jax.experimental import pallas as pl


def kernel(x, edge_index, batch, W1, b1, W2, b2, p_weight, lin_w, lin_b):
    raise NotImplementedError("write your pallas kernel here")


# speedup vs baseline: 1.0218x; 1.0218x over previous
import math

import jax
import jax.numpy as jnp
from jax.experimental import pallas as pl
from jax.experimental.pallas import tpu as pltpu

N = 10000
E = 320000
D = 128
H = 128
NUM_GRAPHS = 64
RATIO = 0.5

ROWS_PER_BLK = 400


def _matmul_blk(x_ref, w_ref, o_ref):
    o_ref[...] = jnp.dot(x_ref[...], w_ref[...],
                         preferred_element_type=jnp.float32)


def _matmul(x, w):
    n = x.shape[0]
    grid = n // ROWS_PER_BLK
    return pl.pallas_call(
        _matmul_blk,
        grid=(grid,),
        in_specs=[
            pl.BlockSpec((ROWS_PER_BLK, x.shape[1]), lambda i: (i, 0)),
            pl.BlockSpec((w.shape[0], w.shape[1]), lambda i: (0, 0)),
        ],
        out_specs=pl.BlockSpec((ROWS_PER_BLK, w.shape[1]), lambda i: (i, 0)),
        out_shape=jax.ShapeDtypeStruct((n, w.shape[1]), jnp.float32),
    )(x, w)


def _gcn_conv(x, src, dst, W, b, num_nodes, h=None):
    if h is None:
        h = _matmul(x, W)
    loops = jnp.arange(num_nodes, dtype=src.dtype)
    si = jnp.concatenate([src, loops])
    di = jnp.concatenate([dst, loops])
    ones = jnp.ones(si.shape[0], dtype=x.dtype)
    deg = jax.ops.segment_sum(ones, di, num_segments=num_nodes)
    dinv = jnp.where(deg > 0, 1.0 / jnp.sqrt(deg), 0.0)
    norm = dinv[si] * dinv[di]
    msgs = h[si] * norm[:, None]
    out = jax.ops.segment_sum(msgs, di, num_segments=num_nodes)
    return out + b


def _topk(score, batch):
    counts = jax.ops.segment_sum(jnp.ones((N,), jnp.int32), batch,
                                 num_segments=NUM_GRAPHS)
    k = jnp.ceil(RATIO * counts.astype(jnp.float32)).astype(jnp.int32)
    idx = jnp.arange(N, dtype=jnp.int32)
    same = batch[:, None] == batch[None, :]
    better = (score[None, :] > score[:, None]) | (
        (score[None, :] == score[:, None]) & (idx[None, :] < idx[:, None]))
    rank = jnp.sum(same & better, axis=1).astype(jnp.int32)
    keep = rank < k[batch]
    kstart = jnp.cumsum(k) - k
    n_new = jnp.sum(jnp.where(counts > 0, k, 0))
    newpos = kstart[batch] + rank
    remap = jnp.where(keep, newpos, -1)
    perm = jnp.zeros((N,), jnp.int32).at[jnp.where(keep, newpos, N)].set(
        idx, mode='drop')
    valid = idx < n_new
    return perm, remap, valid


def kernel(x, edge_index, batch, W1, b1, W2, b2, p_weight, lin_w, lin_b):
    src, dst = edge_index[0], edge_index[1]
    x1 = _gcn_conv(x, src, dst, W1, b1, N)
    score = jnp.tanh((x1 @ p_weight) / jnp.linalg.norm(p_weight))
    perm, remap, valid = _topk(score, batch)
    keep_e = (remap[src] >= 0) & (remap[dst] >= 0)
    src_p = jnp.where(keep_e, remap[src], N).astype(jnp.int32)
    dst_p = jnp.where(keep_e, remap[dst], N).astype(jnp.int32)
    x_p = x1[perm] * score[perm][:, None]
    x_p = jnp.where(valid[:, None], x_p, 0.0)
    x_p_pad = jnp.concatenate([x_p, jnp.zeros((1, H), dtype=x_p.dtype)])
    x2 = _gcn_conv(x_p_pad, src_p, dst_p, W2, b2, N + 1,
                   h=jnp.concatenate([_matmul(x_p, W2),
                                      jnp.zeros((1, H), jnp.float32)]))[:N]
    x2 = jnp.where(valid[:, None], x2, 0.0)
    batch_p = jnp.where(valid, batch[perm], NUM_GRAPHS)
    pooled = jax.ops.segment_sum(x2, batch_p, num_segments=NUM_GRAPHS)
    out = pooled @ lin_w.T + lin_b
    return out


# speedup vs baseline: 21.4547x; 20.9968x over previous
import functools

import jax
import jax.numpy as jnp
from jax import lax
from jax.experimental import pallas as pl
from jax.experimental.pallas import tpu as pltpu
from jax.experimental.pallas import tpu_sc as plsc

N = 10000
E = 320000
H = 128
NG = 64

NROWS = 10240
JUNK = 256
NW = 32
EPW = E // NW
C = 200
NCHUNK = EPW // C
NFILL = (C + 15) // 16
RPT = NROWS // 16
PSL = (NROWS + JUNK) // 16
SPW = NROWS // NW
BLK = 640
GRID = NROWS // BLK

_mesh = plsc.VectorSubcoreMesh(core_axis_name="c", subcore_axis_name="s")
F32 = jnp.float32
I32 = jnp.int32


def _fill1d(ref, n, value):
    def fill(i, _):
        ref[pl.ds(jnp.minimum(i * 16, n - 16), 16)] = jnp.full((16,), value,
                                                               ref.dtype)
        return 0
    lax.fori_loop(0, (n + 15) // 16, fill, 0)


def _deg_body(dst_hbm, out_hbm, didx, ones, zb, acc):
    c = lax.axis_index("c")
    s = lax.axis_index("s")
    w = c * 16 + s
    _fill1d(zb, RPT, 0.0)
    pltpu.sync_copy(zb, acc.at[pl.ds(s * RPT, RPT)])
    _fill1d(ones, C, 1.0)
    plsc.subcore_barrier()

    def chunk(i, _):
        off = w * EPW + i * C
        pltpu.sync_copy(dst_hbm.at[pl.ds(off, C)], didx)
        pltpu.sync_copy(ones, acc.at[didx], add=True)
        return 0
    lax.fori_loop(0, NCHUNK, chunk, 0)
    plsc.subcore_barrier()
    pltpu.sync_copy(acc.at[pl.ds(s * RPT, RPT)], zb)
    pltpu.sync_copy(zb, out_hbm.at[c, pl.ds(s * RPT, RPT)])


_sc_deg = functools.partial(
    pl.kernel, _deg_body, mesh=_mesh,
    out_type=jax.ShapeDtypeStruct((2, NROWS), F32),
    scratch_types=[
        pltpu.VMEM((C,), I32),
        pltpu.VMEM((C,), F32),
        pltpu.VMEM((RPT,), F32),
        pltpu.VMEM_SHARED((NROWS,), F32),
    ])()


def _mp_body(hp_hbm, src_hbm, dst_hbm, zeros2, out_hbm, sidx, didx, rows, acc,
             sem):
    c = lax.axis_index("c")
    s = lax.axis_index("s")
    w = c * 16 + s
    pltpu.sync_copy(zeros2.at[pl.ds(0, 160)], rows.at[pl.ds(0, 160)])
    for t in range(4):
        pltpu.sync_copy(rows.at[pl.ds(0, 160)],
                        acc.at[pl.ds(s * RPT + t * 160, 160)])
    plsc.subcore_barrier()

    def chunk(i, _):
        off = w * EPW + i * C
        pltpu.sync_copy(src_hbm.at[pl.ds(off, C)], sidx)
        pltpu.sync_copy(dst_hbm.at[pl.ds(off, C)], didx)
        pltpu.async_copy(hp_hbm.at[sidx], rows, sem).wait()
        pltpu.sync_copy(rows, acc.at[didx], add=True)
        return 0
    lax.fori_loop(0, NCHUNK, chunk, 0)
    plsc.subcore_barrier()
    for t in range(4):
        pltpu.sync_copy(acc.at[pl.ds(s * RPT + t * 160, 160)],
                        rows.at[pl.ds(0, 160)])
        pltpu.sync_copy(rows.at[pl.ds(0, 160)],
                        out_hbm.at[c, pl.ds(s * RPT + t * 160, 160)])


_sc_mp = functools.partial(
    pl.kernel, _mp_body, mesh=_mesh,
    out_type=jax.ShapeDtypeStruct((2, NROWS, H), F32),
    scratch_types=[
        pltpu.VMEM((C,), I32),
        pltpu.VMEM((C,), I32),
        pltpu.VMEM((C, H), F32),
        pltpu.VMEM_SHARED((NROWS, H), F32),
        pltpu.SemaphoreType.DMA,
    ])()


def _rm_body(remap_hbm, src_hbm, dst_hbm, x1s_hbm,
             srcp_hbm, dstp_hbm, deg2_hbm, xp_hbm,
             remap_v, srcb, dstb, rsb, rdb, spb, dpb, ones, updb, tgtb, pidx,
             rowsb, pbuf, zb, perm_sh, deg2_sh, sem):
    c = lax.axis_index("c")
    s = lax.axis_index("s")
    w = c * 16 + s
    pltpu.sync_copy(remap_hbm.at[pl.ds(s * RPT, RPT)], remap_v)

    def pfill(i, _):
        o = jnp.minimum(i * 16, PSL - 16)
        slot = s * PSL + o + lax.iota(I32, 16)
        pbuf[pl.ds(o, 16)] = N + (slot & 127)
        return 0
    lax.fori_loop(0, (PSL + 15) // 16, pfill, 0)
    pltpu.sync_copy(pbuf, perm_sh.at[pl.ds(s * PSL, PSL)])
    _fill1d(zb, RPT, 0.0)
    pltpu.sync_copy(zb, deg2_sh.at[pl.ds(s * RPT, RPT)])
    _fill1d(ones, C, 1.0)
    plsc.subcore_barrier()

    base = s * RPT

    def bfill(j, _):
        nid = base + j * 16 + lax.iota(I32, 16)
        rm = remap_v[pl.ds(j * 16, 16)]
        tgt = jnp.where(rm < 0, NROWS + (nid & (JUNK - 1)), rm)
        updb[pl.ds(j * 16, 16)] = nid
        tgtb[pl.ds(j * 16, 16)] = tgt
        return 0
    lax.fori_loop(0, RPT // 16, bfill, 0)
    pltpu.sync_copy(updb, perm_sh.at[tgtb])
    plsc.subcore_barrier()

    jbase = w * SPW
    pltpu.sync_copy(perm_sh.at[pl.ds(jbase, SPW)], pidx)
    pltpu.async_copy(x1s_hbm.at[pidx], rowsb, sem).wait()
    pltpu.sync_copy(rowsb, xp_hbm.at[pl.ds(jbase, SPW)])

    def echunk(i, _):
        off = w * EPW + i * C
        pltpu.sync_copy(src_hbm.at[pl.ds(off, C)], srcb)
        pltpu.sync_copy(dst_hbm.at[pl.ds(off, C)], dstb)
        pltpu.async_copy(remap_hbm.at[srcb], rsb, sem).wait()
        pltpu.async_copy(remap_hbm.at[dstb], rdb, sem).wait()

        def inner(j, _):
            jo = jnp.minimum(j * 16, C - 16)
            rs = rsb[pl.ds(jo, 16)]
            rd = rdb[pl.ds(jo, 16)]
            keep = (rs >= 0) & (rd >= 0)
            eid = off + jo + lax.iota(I32, 16)
            pad = N + (eid & 127)
            spb[pl.ds(jo, 16)] = jnp.where(keep, rs, pad)
            dpb[pl.ds(jo, 16)] = jnp.where(keep, rd, pad)
            return 0
        lax.fori_loop(0, NFILL, inner, 0)
        pltpu.sync_copy(spb, srcp_hbm.at[pl.ds(off, C)])
        pltpu.sync_copy(dpb, dstp_hbm.at[pl.ds(off, C)])
        pltpu.sync_copy(ones, deg2_sh.at[dpb], add=True)
        return 0
    lax.fori_loop(0, NCHUNK, echunk, 0)
    plsc.subcore_barrier()
    pltpu.sync_copy(deg2_sh.at[pl.ds(s * RPT, RPT)], zb)
    pltpu.sync_copy(zb, deg2_hbm.at[c, pl.ds(s * RPT, RPT)])


_sc_rm = functools.partial(
    pl.kernel, _rm_body, mesh=_mesh,
    out_type=(
        jax.ShapeDtypeStruct((E,), I32),
        jax.ShapeDtypeStruct((E,), I32),
        jax.ShapeDtypeStruct((2, NROWS), F32),
        jax.ShapeDtypeStruct((NROWS, H), F32),
    ),
    scratch_types=[
        pltpu.VMEM((RPT,), I32),
        pltpu.VMEM((C,), I32),
        pltpu.VMEM((C,), I32),
        pltpu.VMEM((C,), I32),
        pltpu.VMEM((C,), I32),
        pltpu.VMEM((C,), I32),
        pltpu.VMEM((C,), I32),
        pltpu.VMEM((C,), F32),
        pltpu.VMEM((RPT,), I32),
        pltpu.VMEM((RPT,), I32),
        pltpu.VMEM((SPW,), I32),
        pltpu.VMEM((SPW, H), F32),
        pltpu.VMEM((PSL,), I32),
        pltpu.VMEM((RPT,), F32),
        pltpu.VMEM_SHARED((NROWS + JUNK,), I32),
        pltpu.VMEM_SHARED((NROWS,), F32),
        pltpu.SemaphoreType.DMA,
    ])()


def _ta_body(x_ref, w_ref, degt_ref, h_ref, hp_ref, dinv_ref):
    h = jnp.dot(x_ref[...], w_ref[...], preferred_element_type=F32)
    deg = degt_ref[...]
    dinv = lax.rsqrt(deg[:, 0:1] + deg[:, 1:2] + 1.0)
    h_ref[...] = h
    hp_ref[...] = h * dinv
    dinv_ref[...] = dinv


def _ta(x_pad, W1, degT):
    return pl.pallas_call(
        _ta_body,
        grid=(GRID,),
        in_specs=[
            pl.BlockSpec((BLK, H), lambda i: (i, 0)),
            pl.BlockSpec((H, H), lambda i: (0, 0)),
            pl.BlockSpec((BLK, 2), lambda i: (i, 0)),
        ],
        out_specs=[
            pl.BlockSpec((BLK, H), lambda i: (i, 0)),
            pl.BlockSpec((BLK, H), lambda i: (i, 0)),
            pl.BlockSpec((BLK, 1), lambda i: (i, 0)),
        ],
        out_shape=[
            jax.ShapeDtypeStruct((NROWS, H), F32),
            jax.ShapeDtypeStruct((NROWS, H), F32),
            jax.ShapeDtypeStruct((NROWS, 1), F32),
        ],
    )(x_pad, W1, degT)


def _tb1_body(s1_ref, h_ref, dinv_ref, b1_ref, pw_ref, x1s_ref, sc_ref):
    i = pl.program_id(0)
    sp = s1_ref[...]
    S = sp[0] + sp[1]
    dinv = dinv_ref[...]
    x1 = dinv * S + dinv * dinv * h_ref[...] + b1_ref[...]
    row = i * BLK + lax.broadcasted_iota(I32, (BLK, 1), 0)
    x1 = jnp.where(row < N, x1, 0.0)
    pw = pw_ref[...]
    nrm = jnp.sqrt(jnp.sum(pw * pw))
    sc = jnp.tanh(jnp.dot(x1, pw, preferred_element_type=F32) / nrm)
    sc_ref[...] = sc
    x1s_ref[...] = x1 * sc


def _tb1(s1p, h_pad, dinv, b1r, pwc):
    return pl.pallas_call(
        _tb1_body,
        grid=(GRID,),
        in_specs=[
            pl.BlockSpec((2, BLK, H), lambda i: (0, i, 0)),
            pl.BlockSpec((BLK, H), lambda i: (i, 0)),
            pl.BlockSpec((BLK, 1), lambda i: (i, 0)),
            pl.BlockSpec((1, H), lambda i: (0, 0)),
            pl.BlockSpec((H, 1), lambda i: (0, 0)),
        ],
        out_specs=[
            pl.BlockSpec((BLK, H), lambda i: (i, 0)),
            pl.BlockSpec((BLK, 1), lambda i: (i, 0)),
        ],
        out_shape=[
            jax.ShapeDtypeStruct((NROWS, H), F32),
            jax.ShapeDtypeStruct((NROWS, 1), F32),
        ],
    )(s1p, h_pad, dinv, b1r, pwc)


_WND = 2048
_NWND = NROWS // _WND


def _tb2_body(scc_ref, bc_ref, scr_ref, br_ref, remap_ref):
    i = pl.program_id(0)
    sc_r = scc_ref[...]
    b_r = bc_ref[...]
    idx_r = i * BLK + lax.broadcasted_iota(I32, (BLK, 1), 0)
    rank = jnp.zeros((BLK, 1), F32)
    gc = lax.broadcasted_iota(I32, (NG, 1), 0)
    gr = lax.broadcasted_iota(I32, (1, NG), 1)
    counts = jnp.zeros((NG, 1), F32)
    for wnd in range(_NWND):
        bw = br_ref[pl.ds(wnd, 1), :]
        sw = scr_ref[pl.ds(wnd, 1), :]
        iw = wnd * _WND + lax.broadcasted_iota(I32, (1, _WND), 1)
        same = b_r == bw
        better = (sw > sc_r) | ((sw == sc_r) & (iw < idx_r))
        rank = rank + jnp.sum(jnp.where(same & better, 1.0, 0.0),
                              axis=1, keepdims=True)
        counts = counts + jnp.sum(jnp.where(bw == gc, 1.0, 0.0),
                                  axis=1, keepdims=True)
    k = jnp.floor((counts + 1.0) * 0.5)
    tri = jnp.where(gr < gc, 1.0, 0.0)
    hi = jax.lax.Precision.HIGHEST
    kstart = jnp.floor(jnp.dot(tri, k, preferred_element_type=F32,
                               precision=hi) + 0.5)
    onehot = jnp.where(b_r == gr, 1.0, 0.0)
    k_r = jnp.floor(jnp.dot(onehot, k, preferred_element_type=F32,
                            precision=hi) + 0.5)
    ks_r = jnp.floor(jnp.dot(onehot, kstart, preferred_element_type=F32,
                             precision=hi) + 0.5)
    keep = rank < k_r
    remap_ref[...] = jnp.where(keep, ks_r + rank, -1.0).astype(I32)


def _tb2(score_col, bcol, scr5, br5):
    return pl.pallas_call(
        _tb2_body,
        grid=(GRID,),
        in_specs=[
            pl.BlockSpec((BLK, 1), lambda i: (i, 0)),
            pl.BlockSpec((BLK, 1), lambda i: (i, 0)),
            pl.BlockSpec((_NWND, _WND), lambda i: (0, 0)),
            pl.BlockSpec((_NWND, _WND), lambda i: (0, 0)),
        ],
        out_specs=pl.BlockSpec((BLK, 1), lambda i: (i, 0)),
        out_shape=jax.ShapeDtypeStruct((NROWS, 1), I32),
    )(score_col, bcol, scr5, br5)


def _tc_body(xp_ref, w2_ref, degt_ref, h2_ref, h2p_ref, dinv2_ref):
    h2 = jnp.dot(xp_ref[...], w2_ref[...], preferred_element_type=F32)
    deg = degt_ref[...]
    dinv2 = lax.rsqrt(deg[:, 0:1] + deg[:, 1:2] + 1.0)
    h2_ref[...] = h2
    h2p_ref[...] = h2 * dinv2
    dinv2_ref[...] = dinv2


def _tc(xp, W2, deg2T):
    return pl.pallas_call(
        _tc_body,
        grid=(GRID,),
        in_specs=[
            pl.BlockSpec((BLK, H), lambda i: (i, 0)),
            pl.BlockSpec((H, H), lambda i: (0, 0)),
            pl.BlockSpec((BLK, 2), lambda i: (i, 0)),
        ],
        out_specs=[
            pl.BlockSpec((BLK, H), lambda i: (i, 0)),
            pl.BlockSpec((BLK, H), lambda i: (i, 0)),
            pl.BlockSpec((BLK, 1), lambda i: (i, 0)),
        ],
        out_shape=[
            jax.ShapeDtypeStruct((NROWS, H), F32),
            jax.ShapeDtypeStruct((NROWS, H), F32),
            jax.ShapeDtypeStruct((NROWS, 1), F32),
        ],
    )(xp, W2, deg2T)


def _te_body(s2_ref, h2_ref, dinv2_ref, b2_ref, bc_ref, lw_ref, lb_ref,
             out_ref, acc_ref):
    i = pl.program_id(0)
    sp = s2_ref[...]
    S2 = sp[0] + sp[1]
    dinv2 = dinv2_ref[...]
    x2 = dinv2 * S2 + dinv2 * dinv2 * h2_ref[...] + b2_ref[...]
    bc = bc_ref[...]
    gr = lax.broadcasted_iota(I32, (1, NG), 1)
    cmp = jnp.where(bc == gr, 1.0, 0.0)
    counts = jnp.sum(cmp, axis=0, keepdims=True)
    k = jnp.floor((counts + 1.0) * 0.5)
    ga = lax.broadcasted_iota(I32, (NG, 1), 0)
    gb = lax.broadcasted_iota(I32, (1, NG), 1)
    tri = jnp.where(ga < gb, 1.0, 0.0)
    hi = jax.lax.Precision.HIGHEST
    kstart = jnp.floor(jnp.dot(k, tri, preferred_element_type=F32,
                               precision=hi) + 0.5)
    kend = kstart + k
    jcol = i * BLK + lax.broadcasted_iota(I32, (BLK, 1), 0)
    jf = jcol.astype(F32)
    M = jnp.where((jf >= kstart) & (jf < kend), 1.0, 0.0)
    contrib = lax.dot_general(M, x2, (((0,), (0,)), ((), ())),
                              preferred_element_type=F32, precision=hi)

    @pl.when(i == 0)
    def _():
        acc_ref[...] = jnp.zeros((NG, H), F32)

    acc_ref[...] += contrib

    @pl.when(i == GRID - 1)
    def _():
        out_ref[...] = (jnp.dot(acc_ref[...], lw_ref[...],
                                preferred_element_type=F32,
                                precision=jax.lax.Precision.HIGHEST)
                        + lb_ref[...])


def _te(s2p, h2, dinv2, b2r, bcol, lwc, lbr):
    return pl.pallas_call(
        _te_body,
        grid=(GRID,),
        in_specs=[
            pl.BlockSpec((2, BLK, H), lambda i: (0, i, 0)),
            pl.BlockSpec((BLK, H), lambda i: (i, 0)),
            pl.BlockSpec((BLK, 1), lambda i: (i, 0)),
            pl.BlockSpec((1, H), lambda i: (0, 0)),
            pl.BlockSpec((NROWS, 1), lambda i: (0, 0)),
            pl.BlockSpec((H, 1), lambda i: (0, 0)),
            pl.BlockSpec((1, 1), lambda i: (0, 0)),
        ],
        out_specs=pl.BlockSpec((NG, 1), lambda i: (0, 0)),
        out_shape=jax.ShapeDtypeStruct((NG, 1), F32),
        scratch_shapes=[pltpu.VMEM((NG, H), F32)],
    )(s2p, h2, dinv2, b2r, bcol, lwc, lbr)


def kernel(x, edge_index, batch, W1, b1, W2, b2, p_weight, lin_w, lin_b):
    src = edge_index[0]
    dst = edge_index[1]
    x_pad = jnp.pad(x, ((0, NROWS - N), (0, 0)))
    zeros2 = jnp.zeros((NROWS, H), F32)
    batch_pad = jnp.pad(batch.astype(I32), (0, NROWS - N),
                        constant_values=127)
    bcol = batch_pad.reshape(NROWS, 1)
    brow = batch_pad.reshape(1, NROWS)

    deg1p = _sc_deg(dst)
    h_pad, hp_pad, dinv = _ta(x_pad, W1, deg1p.T)
    s1p = _sc_mp(hp_pad, src, dst, zeros2)
    x1s, score_col = _tb1(s1p, h_pad, dinv, b1.reshape(1, H),
                          p_weight.reshape(H, 1))
    remap_col = _tb2(score_col, bcol, score_col.reshape(_NWND, _WND),
                     batch_pad.reshape(_NWND, _WND))
    srcp, dstp, deg2p, xp = _sc_rm(remap_col.reshape(NROWS), src, dst, x1s)
    h2, h2p, dinv2 = _tc(xp, W2, deg2p.T)
    s2p = _sc_mp(h2p, srcp, dstp, zeros2)
    out = _te(s2p, h2, dinv2, b2.reshape(1, H), bcol,
              lin_w.reshape(H, 1), lin_b.reshape(1, 1))
    return out
